# baseline (speedup 1.0000x reference)
import functools

import numpy as np
import jax
import jax.numpy as jnp
from jax import lax
from jax.experimental import pallas as pl
from jax.experimental.pallas import tpu as pltpu
from jax.experimental.pallas import tpu_sc as plsc

V = 1000
VP = 1024
D = 128
B = 4096
L = 20
NW = 32
BPW = B // NW
K = 32
LANES = 16
BBLK = 256


def _mz_body(emb_ref, lw_ref, b_ref, m_ref, z_ref):
    m = lax.dot_general(emb_ref[...], lw_ref[...],
                        (((1,), (1,)), ((), ())),
                        preferred_element_type=jnp.float32)
    m = m + b_ref[...]
    m_ref[:, :V] = m
    rmax = jnp.max(m, axis=1, keepdims=True)
    s = jnp.sum(jnp.exp(m - rmax), axis=1, keepdims=True)
    z_ref[...] = rmax + jnp.log(s)


def _compute_m_z(emb_table, lin_w, lin_b):
    bias = lin_b.reshape(1, V)
    return pl.pallas_call(
        _mz_body,
        out_shape=[
            jax.ShapeDtypeStruct((V, VP), jnp.float32),
            jax.ShapeDtypeStruct((V, 1), jnp.float32),
        ],
    )(emb_table, lin_w, bias)


def _sc_body(m_hbm, centers_hbm, ctx_hbm, z_hbm, rtab_hbm, part_hbm,
             c_v, ctx_v, z_v, idx_v, rtab_v, buf_v, acc1, acc2, sem):
    cid = lax.axis_index("c")
    sid = lax.axis_index("s")
    wid = sid * 2 + cid
    b0w = wid * BPW

    pltpu.sync_copy(centers_hbm.at[pl.ds(b0w, BPW)], c_v)
    pltpu.sync_copy(ctx_hbm.at[pl.ds(b0w * L, BPW * L)], ctx_v)
    pltpu.sync_copy(z_hbm.at[:], z_v)
    pltpu.sync_copy(rtab_hbm.at[:], rtab_v)

    acc1[...] = jnp.zeros((LANES,), jnp.float32)
    acc2[...] = jnp.zeros((LANES,), jnp.float32)

    for ci in range(BPW // K):
        b0l = ci * K
        for j in range(K // LANES):
            idx_v[pl.ds(j * LANES, LANES)] = c_v[pl.ds(b0l + j * LANES,
                                                       LANES)]
        pltpu.async_copy(m_hbm.at[idx_v], buf_v, sem).wait()
        for t in range(K * L // LANES):
            row = rtab_v[pl.ds(t * LANES, LANES)]
            col = ctx_v[pl.ds(b0l * L + t * LANES, LANES)]
            acc2[...] = acc2[...] + plsc.load_gather(buf_v, [row, col])

    for j in range(BPW // LANES):
        acc1[...] = acc1[...] + plsc.load_gather(
            z_v, [c_v[pl.ds(j * LANES, LANES)]])

    acc1[...] = acc1[...] * jnp.float32(L) - acc2[...]
    pltpu.sync_copy(acc1, part_hbm.at[wid])


@functools.partial(
    pl.kernel,
    out_type=[
        jax.ShapeDtypeStruct((NW, LANES), jnp.float32),
    ],
    mesh=plsc.VectorSubcoreMesh(core_axis_name="c", subcore_axis_name="s"),
    compiler_params=pltpu.CompilerParams(needs_layout_passes=False),
    scratch_types=[
        pltpu.VMEM((BPW,), jnp.int32),
        pltpu.VMEM((BPW * L,), jnp.int32),
        pltpu.VMEM((V,), jnp.float32),
        pltpu.VMEM((K,), jnp.int32),
        pltpu.VMEM((K * L,), jnp.int32),
        pltpu.VMEM((K, VP), jnp.float32),
        pltpu.VMEM((LANES,), jnp.float32),
        pltpu.VMEM((LANES,), jnp.float32),
        pltpu.SemaphoreType.DMA,
    ],
)
def _sc_loss(m_hbm, centers_hbm, ctx_hbm, z_hbm, rtab_hbm, part_hbm,
             *scratch):
    _sc_body(m_hbm, centers_hbm, ctx_hbm, z_hbm, rtab_hbm, part_hbm,
             *scratch)


def _out_body(lw_ref, et_ref, bc_ref, c_ref, out_ref, mt_s):
    i = pl.program_id(0)

    @pl.when(i == 0)
    def _():
        mt = lax.dot_general(lw_ref[...], et_ref[...],
                             (((1,), (1,)), ((), ())),
                             preferred_element_type=jnp.float32)
        mt_s[...] = mt + bc_ref[...]

    c = c_ref[0, 0, :]
    u = lax.broadcasted_iota(jnp.int32, (V, BBLK), 0)
    oh = (u == c[None, :]).astype(jnp.float32)
    pt = jnp.dot(mt_s[...], oh,
                 preferred_element_type=jnp.float32)
    out_ref[...] = jnp.broadcast_to(pt[None, :, :], (L, V, BBLK))


def _compute_out_t(lin_w, emb_table, lin_b, centers):
    bcol = lin_b.reshape(V, 1)
    c3 = centers.reshape(B // BBLK, 1, BBLK)
    return pl.pallas_call(
        _out_body,
        grid=(B // BBLK,),
        in_specs=[
            pl.BlockSpec((V, D), lambda i: (0, 0)),
            pl.BlockSpec((V, D), lambda i: (0, 0)),
            pl.BlockSpec((V, 1), lambda i: (0, 0)),
            pl.BlockSpec((1, 1, BBLK), lambda i: (i, 0, 0)),
        ],
        out_specs=pl.BlockSpec((L, V, BBLK), lambda i: (0, 0, i)),
        out_shape=jax.ShapeDtypeStruct((L, V, B), jnp.float32),
        scratch_shapes=[pltpu.VMEM((V, V), jnp.float32)],
    )(lin_w, emb_table, bcol, c3)


def kernel(centers, contexts, emb_table, lin_w, lin_b):
    centers = centers.astype(jnp.int32)
    m, z = _compute_m_z(emb_table, lin_w, lin_b)
    rtab = jnp.asarray(np.arange(K * L, dtype=np.int32) // L)
    (part,) = _sc_loss(
        m,
        centers,
        contexts.astype(jnp.int32).reshape(B * L),
        z.reshape(V),
        rtab,
    )
    xt = _compute_out_t(lin_w, emb_table, lin_b, centers)
    p_ctx = jnp.transpose(xt, (2, 0, 1))
    loss = jnp.sum(part) / jnp.float32(B * L)
    return p_ctx, loss


# speedup vs baseline: 1.0033x; 1.0033x over previous
import functools

import numpy as np
import jax
import jax.numpy as jnp
from jax import lax
from jax.experimental import pallas as pl
from jax.experimental.pallas import tpu as pltpu
from jax.experimental.pallas import tpu_sc as plsc

V = 1000
VP = 1024
D = 128
B = 4096
L = 20
NW = 32
BPW = B // NW
K = 32
LANES = 16
BBLK = 128


def _mz_body(emb_ref, lw_ref, b_ref, m_ref, z_ref):
    m = lax.dot_general(emb_ref[...], lw_ref[...],
                        (((1,), (1,)), ((), ())),
                        preferred_element_type=jnp.float32)
    m = m + b_ref[...]
    m_ref[:, :V] = m
    rmax = jnp.max(m, axis=1, keepdims=True)
    s = jnp.sum(jnp.exp(m - rmax), axis=1, keepdims=True)
    z_ref[...] = rmax + jnp.log(s)


def _compute_m_z(emb_table, lin_w, lin_b):
    bias = lin_b.reshape(1, V)
    return pl.pallas_call(
        _mz_body,
        out_shape=[
            jax.ShapeDtypeStruct((V, VP), jnp.float32),
            jax.ShapeDtypeStruct((V, 1), jnp.float32),
        ],
    )(emb_table, lin_w, bias)


def _sc_body(m_hbm, centers_hbm, ctx_hbm, z_hbm, rtab_hbm, part_hbm,
             c_v, ctx_v, z_v, idx_v, rtab_v, buf_v, acc1, acc2, sem):
    cid = lax.axis_index("c")
    sid = lax.axis_index("s")
    wid = sid * 2 + cid
    b0w = wid * BPW

    pltpu.sync_copy(centers_hbm.at[pl.ds(b0w, BPW)], c_v)
    pltpu.sync_copy(ctx_hbm.at[pl.ds(b0w * L, BPW * L)], ctx_v)
    pltpu.sync_copy(z_hbm.at[:], z_v)
    pltpu.sync_copy(rtab_hbm.at[:], rtab_v)

    acc1[...] = jnp.zeros((LANES,), jnp.float32)
    acc2[...] = jnp.zeros((LANES,), jnp.float32)

    for ci in range(BPW // K):
        b0l = ci * K
        for j in range(K // LANES):
            idx_v[pl.ds(j * LANES, LANES)] = c_v[pl.ds(b0l + j * LANES,
                                                       LANES)]
        pltpu.async_copy(m_hbm.at[idx_v], buf_v, sem).wait()
        for t in range(K * L // LANES):
            row = rtab_v[pl.ds(t * LANES, LANES)]
            col = ctx_v[pl.ds(b0l * L + t * LANES, LANES)]
            acc2[...] = acc2[...] + plsc.load_gather(buf_v, [row, col])

    for j in range(BPW // LANES):
        acc1[...] = acc1[...] + plsc.load_gather(
            z_v, [c_v[pl.ds(j * LANES, LANES)]])

    acc1[...] = acc1[...] * jnp.float32(L) - acc2[...]
    pltpu.sync_copy(acc1, part_hbm.at[wid])


@functools.partial(
    pl.kernel,
    out_type=[
        jax.ShapeDtypeStruct((NW, LANES), jnp.float32),
    ],
    mesh=plsc.VectorSubcoreMesh(core_axis_name="c", subcore_axis_name="s"),
    compiler_params=pltpu.CompilerParams(needs_layout_passes=False),
    scratch_types=[
        pltpu.VMEM((BPW,), jnp.int32),
        pltpu.VMEM((BPW * L,), jnp.int32),
        pltpu.VMEM((V,), jnp.float32),
        pltpu.VMEM((K,), jnp.int32),
        pltpu.VMEM((K * L,), jnp.int32),
        pltpu.VMEM((K, VP), jnp.float32),
        pltpu.VMEM((LANES,), jnp.float32),
        pltpu.VMEM((LANES,), jnp.float32),
        pltpu.SemaphoreType.DMA,
    ],
)
def _sc_loss(m_hbm, centers_hbm, ctx_hbm, z_hbm, rtab_hbm, part_hbm,
             *scratch):
    _sc_body(m_hbm, centers_hbm, ctx_hbm, z_hbm, rtab_hbm, part_hbm,
             *scratch)


def _out_body(lw_ref, et_ref, bc_ref, c_ref, out_ref, mt_s):
    i = pl.program_id(0)

    @pl.when(i == 0)
    def _():
        mt = lax.dot_general(lw_ref[...], et_ref[...],
                             (((1,), (1,)), ((), ())),
                             preferred_element_type=jnp.float32)
        mt_s[...] = mt + bc_ref[...]

    c = c_ref[0, 0, :]
    u = lax.broadcasted_iota(jnp.int32, (V, BBLK), 0)
    oh = (u == c[None, :]).astype(jnp.float32)
    pt = jnp.dot(mt_s[...], oh,
                 preferred_element_type=jnp.float32)
    out_ref[...] = jnp.broadcast_to(pt[None, :, :], (L, V, BBLK))


def _compute_out_t(lin_w, emb_table, lin_b, centers):
    bcol = lin_b.reshape(V, 1)
    c3 = centers.reshape(B // BBLK, 1, BBLK)
    return pl.pallas_call(
        _out_body,
        grid=(B // BBLK,),
        in_specs=[
            pl.BlockSpec((V, D), lambda i: (0, 0)),
            pl.BlockSpec((V, D), lambda i: (0, 0)),
            pl.BlockSpec((V, 1), lambda i: (0, 0)),
            pl.BlockSpec((1, 1, BBLK), lambda i: (i, 0, 0)),
        ],
        out_specs=pl.BlockSpec((L, V, BBLK), lambda i: (0, 0, i)),
        out_shape=jax.ShapeDtypeStruct((L, V, B), jnp.float32),
        scratch_shapes=[pltpu.VMEM((V, V), jnp.float32)],
    )(lin_w, emb_table, bcol, c3)


def kernel(centers, contexts, emb_table, lin_w, lin_b):
    centers = centers.astype(jnp.int32)
    m, z = _compute_m_z(emb_table, lin_w, lin_b)
    rtab = jnp.asarray(np.arange(K * L, dtype=np.int32) // L)
    (part,) = _sc_loss(
        m,
        centers,
        contexts.astype(jnp.int32).reshape(B * L),
        z.reshape(V),
        rtab,
    )
    xt = _compute_out_t(lin_w, emb_table, lin_b, centers)
    p_ctx = jnp.transpose(xt, (2, 0, 1))
    loss = jnp.sum(part) / jnp.float32(B * L)
    return p_ctx, loss
